# baseline (speedup 1.0000x reference)
import functools

import jax
import jax.numpy as jnp
import numpy as np
from jax import lax
from jax.experimental import pallas as pl
from jax.experimental.pallas import tpu as pltpu
from jax.experimental.pallas import tpu_sc as plsc

BATCH = 128
FEAT = 32768
KEEP = 8192
OUT_SCALE = 4.0

NC = 2
NS = 16
L = 16
NW = NC * NS
ROWS_PER_W = BATCH // NW
NVREG = FEAT // L
NBUCKET = 256
NSUB = 2
HWORDS = NBUCKET * L + L


def _f32_to_key(x):
    b = lax.bitcast_convert_type(x, jnp.int32)
    m = lax.shift_right_arithmetic(b, 31)
    k = b ^ (m | jnp.int32(-2147483648))
    return lax.bitcast_convert_type(k, jnp.uint32)


def _key_to_f32(k):
    ki = lax.bitcast_convert_type(k, jnp.int32)
    mk = lax.shift_right_arithmetic(ki, 31)
    xm = (mk ^ jnp.int32(-1)) | jnp.int32(-2147483648)
    return lax.bitcast_convert_type(ki ^ xm, jnp.float32)


def _maxout_body(feat_hbm, out_hbm, row2_v, key_v, hist_v, scan_v,
                 sem_in, sem_out):
    wid = lax.axis_index("s") * NC + lax.axis_index("c")
    lanes = lax.iota(jnp.int32, L)
    ones = jnp.ones((L,), jnp.int32)
    laneoff = [lanes + sub * HWORDS for sub in range(NSUB)]
    r0 = wid * ROWS_PER_W

    lane15 = lanes * L + (L - 1)

    def _pick(tvec, tgt, scan_v):
        t_rev = jnp.flip(tvec)
        cum = plsc.cumsum(t_rev)
        hit = cum >= tgt
        j = plsc.all_reduce_ffs(hit)
        su = cum - t_rev
        scan_v[pl.ds(0, L)] = su
        s_above = plsc.load_gather(scan_v, [j])
        return 15 - j, tgt - s_above

    def merge_scan(target, scan_v):
        def gbody(g, _):
            acc = jnp.zeros((L,), jnp.int32)
            for j in range(16):
                off = g * (16 * L) + j * L
                for sub in range(NSUB):
                    acc = acc + hist_v[pl.ds(sub * HWORDS + off, L)]
            scan_v[pl.ds(L + g * L, L)] = plsc.cumsum(acc)
            return 0

        lax.fori_loop(0, 16, gbody, 0, unroll=2)
        gt = plsc.load_gather(scan_v, [L + lane15])
        gsel, ksel = _pick(gt, target, scan_v)
        gsel_s = jnp.max(gsel)

        def bbody(b, _):
            off = gsel_s * (16 * L) + b * L
            acc2 = hist_v[pl.ds(off, L)]
            for sub in range(1, NSUB):
                acc2 = acc2 + hist_v[pl.ds(sub * HWORDS + off, L)]
            scan_v[pl.ds(L + b * L, L)] = plsc.cumsum(acc2)
            return 0

        lax.fori_loop(0, 16, bbody, 0, unroll=2)
        bt = plsc.load_gather(scan_v, [L + lane15])
        bsel, ksel2 = _pick(bt, ksel, scan_v)
        return gsel * 16 + bsel, ksel2

    def zero_hist():
        @plsc.parallel_loop(0, NSUB * (NBUCKET + 1), 1, unroll=4)
        def _(i):
            hist_v[pl.ds(i * L, L)] = jnp.zeros((L,), jnp.int32)

    pltpu.async_copy(feat_hbm.at[r0], row2_v.at[pl.ds(0, FEAT)], sem_in)

    def row_body(rr, _):
        off = (rr & 1) * FEAT
        noff = FEAT - off
        zero_hist()
        pltpu.make_async_copy(
            feat_hbm.at[r0], row2_v.at[pl.ds(off, FEAT)], sem_in).wait()

        @plsc.parallel_loop(0, NVREG, NSUB, unroll=4)
        def _(i):
            for sub in range(NSUB):
                base = off + (i + sub) * L
                x = row2_v[pl.ds(base, L)]
                k = _f32_to_key(x)
                key_v[pl.ds((i + sub) * L, L)] = k
                idx16 = (lax.shift_right_logical(k, jnp.uint32(20))
                         & jnp.uint32(0xFF0))
                plsc.addupdate_scatter(
                    hist_v, [idx16.astype(jnp.int32) + laneoff[sub]], ones)

        @pl.when(rr > 0)
        def _():
            pltpu.make_async_copy(
                row2_v.at[pl.ds(noff, FEAT)], out_hbm.at[r0], sem_out).wait()

        @pl.when(rr < ROWS_PER_W - 1)
        def _():
            pltpu.async_copy(
                feat_hbm.at[r0 + rr + 1], row2_v.at[pl.ds(noff, FEAT)],
                sem_in)

        target = jnp.full((L,), KEEP, jnp.int32)
        dsel, target = merge_scan(target, scan_v)
        prefix = dsel.astype(jnp.uint32) << jnp.uint32(24)

        for level in range(1, 3):
            shift_d = 24 - 8 * level
            zero_hist()

            @plsc.parallel_loop(0, NVREG, NSUB, unroll=4)
            def _(i, prefix=prefix, shift_d=shift_d):
                for sub in range(NSUB):
                    k = key_v[pl.ds((i + sub) * L, L)]
                    rel = lax.shift_right_logical(
                        k - prefix, jnp.uint32(shift_d - 4))
                    t = jnp.minimum(rel, jnp.uint32(0x1000))
                    idx16 = t & jnp.uint32(0x1FF0)
                    plsc.addupdate_scatter(
                        hist_v, [idx16.astype(jnp.int32) + laneoff[sub]], ones)

            dsel, target = merge_scan(target, scan_v)
            prefix = prefix | (dsel.astype(jnp.uint32) << jnp.uint32(shift_d))

        thresh = prefix

        @plsc.parallel_loop(0, NVREG, 1, unroll=4)
        def _(i, thresh=thresh):
            k = key_v[pl.ds(i * L, L)]
            x = _key_to_f32(k)
            y = jnp.where(k >= thresh, x * OUT_SCALE, 0.0)
            row2_v[pl.ds(off + i * L, L)] = y

        pltpu.async_copy(
            row2_v.at[pl.ds(off, FEAT)], out_hbm.at[r0 + rr], sem_out)
        return 0

    lax.fori_loop(0, ROWS_PER_W, row_body, 0)
    pltpu.make_async_copy(
        row2_v.at[pl.ds(0, FEAT)], out_hbm.at[r0], sem_out).wait()


@jax.jit
def _maxout_sc(feat):
    mesh = plsc.VectorSubcoreMesh(core_axis_name="c", subcore_axis_name="s")
    f = functools.partial(
        pl.kernel,
        out_type=jax.ShapeDtypeStruct((BATCH, FEAT), jnp.float32),
        mesh=mesh,
        scratch_types=[
            pltpu.VMEM((2 * FEAT,), jnp.float32),
            pltpu.VMEM((FEAT,), jnp.uint32),
            pltpu.VMEM((NSUB * HWORDS,), jnp.int32),
            pltpu.VMEM((L + 16 * L,), jnp.int32),
            pltpu.SemaphoreType.DMA,
            pltpu.SemaphoreType.DMA,
        ],
        compiler_params=pltpu.CompilerParams(needs_layout_passes=False),
    )(_maxout_body)
    return f(feat)


def kernel(feat):
    return _maxout_sc(feat)


# speedup vs baseline: 1.0543x; 1.0543x over previous
import functools

import jax
import jax.numpy as jnp
import numpy as np
from jax import lax
from jax.experimental import pallas as pl
from jax.experimental.pallas import tpu as pltpu
from jax.experimental.pallas import tpu_sc as plsc

BATCH = 128
FEAT = 32768
KEEP = 8192
OUT_SCALE = 4.0

NC = 2
NS = 16
L = 16
NW = NC * NS
ROWS_PER_W = BATCH // NW
NVREG = FEAT // L
NBUCKET = 256
NSUB = 1
HWORDS = NBUCKET * L + L


def _f32_to_key(x):
    b = lax.bitcast_convert_type(x, jnp.int32)
    m = lax.shift_right_arithmetic(b, 31)
    k = b ^ (m | jnp.int32(-2147483648))
    return lax.bitcast_convert_type(k, jnp.uint32)


def _key_to_f32(k):
    ki = lax.bitcast_convert_type(k, jnp.int32)
    mk = lax.shift_right_arithmetic(ki, 31)
    xm = (mk ^ jnp.int32(-1)) | jnp.int32(-2147483648)
    return lax.bitcast_convert_type(ki ^ xm, jnp.float32)


def _maxout_body(feat_hbm, out_hbm, row2_v, key_v, hist_v, scan_v,
                 sem_in, sem_out):
    wid = lax.axis_index("s") * NC + lax.axis_index("c")
    lanes = lax.iota(jnp.int32, L)
    ones = jnp.ones((L,), jnp.int32)
    laneoff = [lanes + sub * HWORDS for sub in range(NSUB)]
    r0 = wid * ROWS_PER_W

    lane15 = lanes * L + (L - 1)

    def _pick(tvec, tgt, scan_v):
        t_rev = jnp.flip(tvec)
        cum = plsc.cumsum(t_rev)
        hit = cum >= tgt
        j = plsc.all_reduce_ffs(hit)
        su = cum - t_rev
        scan_v[pl.ds(0, L)] = su
        s_above = plsc.load_gather(scan_v, [j])
        return 15 - j, tgt - s_above

    def merge_scan(target, scan_v):
        def gbody(g, _):
            acc = jnp.zeros((L,), jnp.int32)
            for j in range(16):
                off = g * (16 * L) + j * L
                for sub in range(NSUB):
                    acc = acc + hist_v[pl.ds(sub * HWORDS + off, L)]
            scan_v[pl.ds(L + g * L, L)] = plsc.cumsum(acc)
            return 0

        lax.fori_loop(0, 16, gbody, 0, unroll=2)
        gt = plsc.load_gather(scan_v, [L + lane15])
        gsel, ksel = _pick(gt, target, scan_v)
        gsel_s = jnp.max(gsel)

        def bbody(b, _):
            off = gsel_s * (16 * L) + b * L
            acc2 = hist_v[pl.ds(off, L)]
            for sub in range(1, NSUB):
                acc2 = acc2 + hist_v[pl.ds(sub * HWORDS + off, L)]
            scan_v[pl.ds(L + b * L, L)] = plsc.cumsum(acc2)
            return 0

        lax.fori_loop(0, 16, bbody, 0, unroll=2)
        bt = plsc.load_gather(scan_v, [L + lane15])
        bsel, ksel2 = _pick(bt, ksel, scan_v)
        return gsel * 16 + bsel, ksel2

    def zero_hist():
        @plsc.parallel_loop(0, NSUB * (NBUCKET + 1), 1, unroll=4)
        def _(i):
            hist_v[pl.ds(i * L, L)] = jnp.zeros((L,), jnp.int32)

    pltpu.async_copy(feat_hbm.at[r0], row2_v.at[pl.ds(0, FEAT)], sem_in)

    def row_body(rr, _):
        off = (rr & 1) * FEAT
        noff = FEAT - off
        zero_hist()
        pltpu.make_async_copy(
            feat_hbm.at[r0], row2_v.at[pl.ds(off, FEAT)], sem_in).wait()

        @plsc.parallel_loop(0, NVREG, NSUB, unroll=4)
        def _(i):
            for sub in range(NSUB):
                base = off + (i + sub) * L
                x = row2_v[pl.ds(base, L)]
                k = _f32_to_key(x)
                key_v[pl.ds((i + sub) * L, L)] = k
                idx16 = (lax.shift_right_logical(k, jnp.uint32(20))
                         & jnp.uint32(0xFF0))
                plsc.addupdate_scatter(
                    hist_v, [idx16.astype(jnp.int32) + laneoff[sub]], ones)

        @pl.when(rr > 0)
        def _():
            pltpu.make_async_copy(
                row2_v.at[pl.ds(noff, FEAT)], out_hbm.at[r0], sem_out).wait()

        @pl.when(rr < ROWS_PER_W - 1)
        def _():
            pltpu.async_copy(
                feat_hbm.at[r0 + rr + 1], row2_v.at[pl.ds(noff, FEAT)],
                sem_in)

        target = jnp.full((L,), KEEP, jnp.int32)
        dsel, target = merge_scan(target, scan_v)
        prefix = dsel.astype(jnp.uint32) << jnp.uint32(24)

        for level in range(1, 3):
            shift_d = 24 - 8 * level
            zero_hist()

            @plsc.parallel_loop(0, NVREG, NSUB, unroll=4)
            def _(i, prefix=prefix, shift_d=shift_d):
                for sub in range(NSUB):
                    k = key_v[pl.ds((i + sub) * L, L)]
                    rel = lax.shift_right_logical(
                        k - prefix, jnp.uint32(shift_d - 4))
                    t = jnp.minimum(rel, jnp.uint32(0x1000))
                    idx16 = t & jnp.uint32(0x1FF0)
                    plsc.addupdate_scatter(
                        hist_v, [idx16.astype(jnp.int32) + laneoff[sub]], ones)

            dsel, target = merge_scan(target, scan_v)
            prefix = prefix | (dsel.astype(jnp.uint32) << jnp.uint32(shift_d))

        thresh = prefix

        @plsc.parallel_loop(0, NVREG, 1, unroll=4)
        def _(i, thresh=thresh):
            k = key_v[pl.ds(i * L, L)]
            x = _key_to_f32(k)
            y = jnp.where(k >= thresh, x * OUT_SCALE, 0.0)
            row2_v[pl.ds(off + i * L, L)] = y

        pltpu.async_copy(
            row2_v.at[pl.ds(off, FEAT)], out_hbm.at[r0 + rr], sem_out)
        return 0

    lax.fori_loop(0, ROWS_PER_W, row_body, 0)
    pltpu.make_async_copy(
        row2_v.at[pl.ds(0, FEAT)], out_hbm.at[r0], sem_out).wait()


@jax.jit
def _maxout_sc(feat):
    mesh = plsc.VectorSubcoreMesh(core_axis_name="c", subcore_axis_name="s")
    f = functools.partial(
        pl.kernel,
        out_type=jax.ShapeDtypeStruct((BATCH, FEAT), jnp.float32),
        mesh=mesh,
        scratch_types=[
            pltpu.VMEM((2 * FEAT,), jnp.float32),
            pltpu.VMEM((FEAT,), jnp.uint32),
            pltpu.VMEM((NSUB * HWORDS,), jnp.int32),
            pltpu.VMEM((L + 16 * L,), jnp.int32),
            pltpu.SemaphoreType.DMA,
            pltpu.SemaphoreType.DMA,
        ],
        compiler_params=pltpu.CompilerParams(needs_layout_passes=False),
    )(_maxout_body)
    return f(feat)


def kernel(feat):
    return _maxout_sc(feat)


# speedup vs baseline: 1.1071x; 1.0501x over previous
import functools

import jax
import jax.numpy as jnp
import numpy as np
from jax import lax
from jax.experimental import pallas as pl
from jax.experimental.pallas import tpu as pltpu
from jax.experimental.pallas import tpu_sc as plsc

BATCH = 128
FEAT = 32768
KEEP = 8192
OUT_SCALE = 4.0

NC = 2
NS = 16
L = 16
NW = NC * NS
ROWS_PER_W = BATCH // NW
NVREG = FEAT // L
NBUCKET = 256
NSUB = 1
HWORDS = NBUCKET * L + L


def _f32_to_key(x):
    b = lax.bitcast_convert_type(x, jnp.int32)
    m = lax.shift_right_arithmetic(b, 31)
    k = b ^ (m | jnp.int32(-2147483648))
    return lax.bitcast_convert_type(k, jnp.uint32)


def _key_to_f32(k):
    ki = lax.bitcast_convert_type(k, jnp.int32)
    mk = lax.shift_right_arithmetic(ki, 31)
    xm = (mk ^ jnp.int32(-1)) | jnp.int32(-2147483648)
    return lax.bitcast_convert_type(ki ^ xm, jnp.float32)


def _maxout_body(feat_hbm, out_hbm, row2_v, key_v, hist_v, scan_v,
                 sem_in, sem_out):
    wid = lax.axis_index("s") * NC + lax.axis_index("c")
    lanes = lax.iota(jnp.int32, L)
    ones = jnp.ones((L,), jnp.int32)
    laneoff = [lanes + sub * HWORDS for sub in range(NSUB)]
    r0 = wid * ROWS_PER_W

    lane15 = lanes * L + (L - 1)

    def _pick(tvec, tgt, scan_v):
        t_rev = jnp.flip(tvec)
        cum = plsc.cumsum(t_rev)
        hit = cum >= tgt
        j = plsc.all_reduce_ffs(hit)
        su = cum - t_rev
        scan_v[pl.ds(0, L)] = su
        s_above = plsc.load_gather(scan_v, [j])
        return 15 - j, tgt - s_above

    def merge_scan(target, scan_v):
        def gbody(g, _):
            acc = jnp.zeros((L,), jnp.int32)
            for j in range(16):
                off = g * (16 * L) + j * L
                for sub in range(NSUB):
                    acc = acc + hist_v[pl.ds(sub * HWORDS + off, L)]
            scan_v[pl.ds(L + g * L, L)] = plsc.cumsum(acc)
            return 0

        lax.fori_loop(0, 16, gbody, 0, unroll=2)
        gt = plsc.load_gather(scan_v, [L + lane15])
        gsel, ksel = _pick(gt, target, scan_v)
        gsel_s = jnp.max(gsel)

        def bbody(b, _):
            off = gsel_s * (16 * L) + b * L
            acc2 = hist_v[pl.ds(off, L)]
            for sub in range(1, NSUB):
                acc2 = acc2 + hist_v[pl.ds(sub * HWORDS + off, L)]
            scan_v[pl.ds(L + b * L, L)] = plsc.cumsum(acc2)
            return 0

        lax.fori_loop(0, 16, bbody, 0, unroll=2)
        bt = plsc.load_gather(scan_v, [L + lane15])
        bsel, ksel2 = _pick(bt, ksel, scan_v)
        return gsel * 16 + bsel, ksel2

    def zero_hist():
        @plsc.parallel_loop(0, NSUB * (NBUCKET + 1), 1, unroll=4)
        def _(i):
            hist_v[pl.ds(i * L, L)] = jnp.zeros((L,), jnp.int32)

    pltpu.async_copy(feat_hbm.at[r0], row2_v.at[pl.ds(0, FEAT)], sem_in)

    def row_body(rr, _):
        off = (rr & 1) * FEAT
        noff = FEAT - off
        zero_hist()
        pltpu.make_async_copy(
            feat_hbm.at[r0], row2_v.at[pl.ds(off, FEAT)], sem_in).wait()

        @plsc.parallel_loop(0, NVREG, NSUB, unroll=8)
        def _(i):
            for sub in range(NSUB):
                base = off + (i + sub) * L
                x = row2_v[pl.ds(base, L)]
                k = _f32_to_key(x)
                key_v[pl.ds((i + sub) * L, L)] = k
                idx16 = (lax.shift_right_logical(k, jnp.uint32(20))
                         & jnp.uint32(0xFF0))
                plsc.addupdate_scatter(
                    hist_v, [idx16.astype(jnp.int32) + laneoff[sub]], ones)

        @pl.when(rr > 0)
        def _():
            pltpu.make_async_copy(
                row2_v.at[pl.ds(noff, FEAT)], out_hbm.at[r0], sem_out).wait()

        @pl.when(rr < ROWS_PER_W - 1)
        def _():
            pltpu.async_copy(
                feat_hbm.at[r0 + rr + 1], row2_v.at[pl.ds(noff, FEAT)],
                sem_in)

        target = jnp.full((L,), KEEP, jnp.int32)
        dsel, target = merge_scan(target, scan_v)
        prefix = dsel.astype(jnp.uint32) << jnp.uint32(24)

        for level in range(1, 3):
            shift_d = 24 - 8 * level
            zero_hist()

            @plsc.parallel_loop(0, NVREG, NSUB, unroll=8)
            def _(i, prefix=prefix, shift_d=shift_d):
                for sub in range(NSUB):
                    k = key_v[pl.ds((i + sub) * L, L)]
                    rel = lax.shift_right_logical(
                        k - prefix, jnp.uint32(shift_d - 4))
                    t = jnp.minimum(rel, jnp.uint32(0x1000))
                    idx16 = t & jnp.uint32(0x1FF0)
                    plsc.addupdate_scatter(
                        hist_v, [idx16.astype(jnp.int32) + laneoff[sub]], ones)

            dsel, target = merge_scan(target, scan_v)
            prefix = prefix | (dsel.astype(jnp.uint32) << jnp.uint32(shift_d))

        thresh = prefix

        @plsc.parallel_loop(0, NVREG, 1, unroll=8)
        def _(i, thresh=thresh):
            k = key_v[pl.ds(i * L, L)]
            x = _key_to_f32(k)
            y = jnp.where(k >= thresh, x * OUT_SCALE, 0.0)
            row2_v[pl.ds(off + i * L, L)] = y

        pltpu.async_copy(
            row2_v.at[pl.ds(off, FEAT)], out_hbm.at[r0 + rr], sem_out)
        return 0

    lax.fori_loop(0, ROWS_PER_W, row_body, 0)
    pltpu.make_async_copy(
        row2_v.at[pl.ds(0, FEAT)], out_hbm.at[r0], sem_out).wait()


@jax.jit
def _maxout_sc(feat):
    mesh = plsc.VectorSubcoreMesh(core_axis_name="c", subcore_axis_name="s")
    f = functools.partial(
        pl.kernel,
        out_type=jax.ShapeDtypeStruct((BATCH, FEAT), jnp.float32),
        mesh=mesh,
        scratch_types=[
            pltpu.VMEM((2 * FEAT,), jnp.float32),
            pltpu.VMEM((FEAT,), jnp.uint32),
            pltpu.VMEM((NSUB * HWORDS,), jnp.int32),
            pltpu.VMEM((L + 16 * L,), jnp.int32),
            pltpu.SemaphoreType.DMA,
            pltpu.SemaphoreType.DMA,
        ],
        compiler_params=pltpu.CompilerParams(needs_layout_passes=False),
    )(_maxout_body)
    return f(feat)


def kernel(feat):
    return _maxout_sc(feat)
